# baseline (speedup 1.0000x reference)
import jax
import jax.numpy as jnp
from jax.experimental import pallas as pl
from jax.experimental.pallas import tpu as pltpu

_E = 8
_TOPK = 2
_BR = 2048


def _router_block(x_ref, w_ref, b_ref, logits_ref, vals_ref, idx_ref):
    x = x_ref[...]
    w = w_ref[...]
    logits = jax.lax.dot_general(
        x, w, (((1,), (1,)), ((), ())), preferred_element_type=jnp.float32
    ) + b_ref[...]
    logits_ref[...] = logits.T

    m1 = jnp.max(logits, axis=-1, keepdims=True)
    i1 = jnp.argmax(logits, axis=-1)
    iota = jax.lax.broadcasted_iota(jnp.int32, logits.shape, 1)
    masked = jnp.where(iota == i1[:, None], -jnp.inf, logits)
    m2 = jnp.max(masked, axis=-1, keepdims=True)
    i2 = jnp.argmax(masked, axis=-1)
    w1 = 1.0 / (1.0 + jnp.exp(m2 - m1))
    vals_ref[...] = jnp.concatenate([w1.T, 1.0 - w1.T], axis=0)
    idx_ref[...] = jnp.concatenate([i1[None, :], i2[None, :]], axis=0)


def kernel(hidden_states, W, b):
    orig_shape = hidden_states.shape
    x = hidden_states.reshape(-1, orig_shape[-1])
    n_tokens, hidden = x.shape
    grid = (n_tokens // _BR,)

    logits_t, vals_t, idx_t = pl.pallas_call(
        _router_block,
        grid=grid,
        in_specs=[
            pl.BlockSpec((_BR, hidden), lambda i: (i, 0)),
            pl.BlockSpec((_E, hidden), lambda i: (0, 0)),
            pl.BlockSpec((1, _E), lambda i: (0, 0)),
        ],
        out_specs=[
            pl.BlockSpec((_E, _BR), lambda i: (0, i)),
            pl.BlockSpec((_TOPK, _BR), lambda i: (0, i)),
            pl.BlockSpec((_TOPK, _BR), lambda i: (0, i)),
        ],
        out_shape=[
            jax.ShapeDtypeStruct((_E, n_tokens), jnp.float32),
            jax.ShapeDtypeStruct((_TOPK, n_tokens), jnp.float32),
            jax.ShapeDtypeStruct((_TOPK, n_tokens), jnp.int32),
        ],
        compiler_params=pltpu.CompilerParams(
            dimension_semantics=("arbitrary",),
        ),
    )(x, W, b.reshape(1, _E))

    return (logits_t.T, vals_t.T, idx_t.T)
